# baseline (speedup 1.0000x reference)
import jax
import jax.numpy as jnp
from jax import lax
from jax.experimental import pallas as pl
from jax.experimental.pallas import tpu as pltpu
from jax.experimental.pallas import tpu_sc as plsc

BATCH = 16384
EMB = 128
NF = 128
OUT_W = EMB + NF

NC = 2
NS = 16
NW = NC * NS
BPW = BATCH // NW
CH = 128
NCH = BPW // CH


def _body(feats_hbm, idx_hbm, w_hbm, out_hbm, idx_v, rows_v, sem):
    wid = lax.axis_index("s") * NC + lax.axis_index("c")
    base = wid * BPW

    pltpu.sync_copy(idx_hbm.at[pl.ds(wid * NCH, NCH)], idx_v)

    copies = []
    for j in range(NCH):
        copies.append(
            pltpu.async_copy(
                w_hbm.at[idx_v.at[j]], rows_v.at[pl.ds(j * CH, CH)], sem
            )
        )

    pltpu.sync_copy(
        feats_hbm.at[pl.ds(base, BPW)],
        out_hbm.at[pl.ds(base, BPW), pl.ds(EMB, NF)],
    )

    for cp in copies:
        cp.wait()

    pltpu.sync_copy(rows_v, out_hbm.at[pl.ds(base, BPW), pl.ds(0, EMB)])


@jax.jit
def _run(feats, idx2, W):
    mesh = plsc.VectorSubcoreMesh(core_axis_name="c", subcore_axis_name="s")
    return pl.kernel(
        _body,
        out_type=jax.ShapeDtypeStruct((BATCH, OUT_W), jnp.float32),
        mesh=mesh,
        scratch_types=[
            pltpu.VMEM((NCH, CH), jnp.int32),
            pltpu.VMEM((BPW, EMB), jnp.float32),
            pltpu.SemaphoreType.DMA,
        ],
        compiler_params=pltpu.CompilerParams(use_tc_tiling_on_sc=False),
    )(feats, idx2, W)


def kernel(x, W):
    idx = x[:, 0].astype(jnp.int32).reshape(NW * NCH, CH)
    return _run(x[:, 1:], idx, W)


# speedup vs baseline: 4.6849x; 4.6849x over previous
import jax
import jax.numpy as jnp
from jax import lax
from jax.experimental import pallas as pl
from jax.experimental.pallas import tpu as pltpu
from jax.experimental.pallas import tpu_sc as plsc

BATCH = 16384
EMB = 128
NF = 128
OUT_W = EMB + NF

NC = 2
NS = 16
NW = NC * NS
BPW = BATCH // NW
CH = 128
NCH = BPW // CH


def _body(feats_hbm, idx_hbm, w_hbm, out_hbm, idx_v, rows_v, f0, f1, sem, fsem):
    wid = lax.axis_index("s") * NC + lax.axis_index("c")
    base = wid * BPW
    fbufs = (f0, f1)

    fcp = [
        pltpu.async_copy(feats_hbm.at[pl.ds(base + j * CH, CH)], fbufs[j], fsem)
        for j in range(2)
    ]

    pltpu.sync_copy(idx_hbm.at[pl.ds(wid * NCH, NCH)], idx_v)

    copies = []
    for j in range(NCH):
        copies.append(
            pltpu.async_copy(
                w_hbm.at[idx_v.at[j]], rows_v.at[pl.ds(j * CH, CH)], sem
            )
        )

    for j in range(NCH):
        fcp[j % 2].wait()
        pltpu.sync_copy(
            fbufs[j % 2],
            out_hbm.at[pl.ds(base + j * CH, CH), pl.ds(EMB, NF)],
        )
        if j + 2 < NCH:
            fcp[j % 2] = pltpu.async_copy(
                feats_hbm.at[pl.ds(base + (j + 2) * CH, CH)], fbufs[j % 2], fsem
            )

    for cp in copies:
        cp.wait()

    pltpu.sync_copy(rows_v, out_hbm.at[pl.ds(base, BPW), pl.ds(0, EMB)])


@jax.jit
def _run(feats, idx2, W):
    mesh = plsc.VectorSubcoreMesh(core_axis_name="c", subcore_axis_name="s")
    return pl.kernel(
        _body,
        out_type=jax.ShapeDtypeStruct((BATCH, OUT_W), jnp.float32),
        mesh=mesh,
        scratch_types=[
            pltpu.VMEM((NCH, CH), jnp.int32),
            pltpu.VMEM((BPW, EMB), jnp.float32),
            pltpu.VMEM((CH, NF), jnp.float32),
            pltpu.VMEM((CH, NF), jnp.float32),
            pltpu.SemaphoreType.DMA,
            pltpu.SemaphoreType.DMA,
        ],
        compiler_params=pltpu.CompilerParams(use_tc_tiling_on_sc=False),
    )(feats, idx2, W)


def kernel(x, W):
    idx = x[:, 0].astype(jnp.int32).reshape(NW * NCH, CH)
    return _run(x[:, 1:], idx, W)
